# baseline (speedup 1.0000x reference)
import functools

import jax
import jax.numpy as jnp
from jax import lax
from jax.experimental import pallas as pl
from jax.experimental.pallas import tpu as pltpu
from jax.experimental.pallas import tpu_sc as plsc

NUM_TOKENS = 16384
SEQ = 26
D = 64
NW = 32
TOK_PER_W = NUM_TOKENS // NW
IDX_PER_W = TOK_PER_W * SEQ
N_T1 = TOK_PER_W // 128
N_UNITS = SEQ * N_T1
NBUF = 4


def _make_kernel():
    mesh = plsc.VectorSubcoreMesh(core_axis_name="c", subcore_axis_name="s")

    @functools.partial(
        pl.kernel,
        mesh=mesh,
        out_type=jax.ShapeDtypeStruct((SEQ, 8, 128, 8, 128), jnp.float32),
        compiler_params=pltpu.CompilerParams(
            use_tc_tiling_on_sc=False, needs_layout_passes=False
        ),
        scratch_types=[
            pltpu.VMEM((IDX_PER_W,), jnp.int32),
            pltpu.VMEM((IDX_PER_W,), jnp.int32),
            pltpu.VMEM((NBUF, 128, D), jnp.float32),
            pltpu.VMEM((NBUF, D, 129), jnp.float32),
            pltpu.SemaphoreType.DMA((NBUF,)),
            pltpu.SemaphoreType.DMA((NBUF,)),
        ],
    )
    def k(idx_hbm, table_hbm, out_hbm, idx_tmp, idx_s, rows, tiles, gsem, osem):
        wid = lax.axis_index("s") * 2 + lax.axis_index("c")
        pltpu.sync_copy(idx_hbm.at[pl.ds(wid * IDX_PER_W, IDX_PER_W)], idx_tmp)

        iota = lax.broadcasted_iota(jnp.int32, (16,), 0)

        def rearr(g, carry):
            t0 = g * 16
            for s in range(SEQ):
                src = iota * SEQ + (t0 * SEQ + s)
                vals = plsc.load_gather(idx_tmp, [src])
                idx_s[pl.ds(s * TOK_PER_W + t0, 16)] = vals
            return carry

        lax.fori_loop(0, TOK_PER_W // 16, rearr, 0)

        def unit_off(u):
            s = u // N_T1
            t1l = lax.rem(u, N_T1)
            return s, t1l, s * TOK_PER_W + t1l * 128

        def gather(u, b):
            _, _, off = unit_off(u)
            return pltpu.make_async_copy(
                table_hbm.at[idx_s.at[pl.ds(off, 128)]],
                rows.at[b],
                gsem.at[b],
            )

        def out_copy(u, b, d1):
            s, t1l, _ = unit_off(u)
            t1g = wid * N_T1 + t1l
            return pltpu.make_async_copy(
                tiles.at[b, pl.ds(d1 * 8, 8), pl.ds(0, 128)],
                out_hbm.at[s, d1, t1g],
                osem.at[b],
            )

        def drain_outs(b):
            pltpu.make_async_copy(
                table_hbm.at[pl.ds(0, 128)], rows.at[b], osem.at[b]
            ).wait()

        d_base = [iota + 16 * kk for kk in range(4)]

        for b in range(NBUF):
            gather(b, b).start()

        def pair(p, carry):
            for b in range(NBUF):
                u = p * NBUF + b
                gather(u, b).wait()

                @pl.when(u >= NBUF)
                def _():
                    drain_outs(b)

                def trans(t2g, c2):
                    for tt in range(8):
                        t2 = t2g * 8 + tt
                        t2v = jnp.full((16,), 0, jnp.int32) + t2
                        for kk in range(4):
                            v = rows[b, t2, pl.ds(16 * kk, 16)]
                            plsc.store_scatter(
                                tiles.at[b], [d_base[kk], t2v], v
                            )
                    return c2

                lax.fori_loop(0, 16, trans, 0)

                @pl.when(u + NBUF < N_UNITS)
                def _():
                    gather(u + NBUF, b).start()

                for d1 in range(8):
                    out_copy(u, b, d1).start()
            return carry

        lax.fori_loop(0, N_UNITS // NBUF, pair, 0)

        for b in range(NBUF):
            drain_outs(b)

    return k


def kernel(token_ids, weights):
    idx = token_ids.astype(jnp.int32).reshape(-1)
    out5 = _make_kernel()(idx, weights)
    return out5.transpose(2, 4, 0, 1, 3).reshape(NUM_TOKENS, SEQ, D)


# speedup vs baseline: 1.0089x; 1.0089x over previous
import functools

import jax
import jax.numpy as jnp
from jax import lax
from jax.experimental import pallas as pl
from jax.experimental.pallas import tpu as pltpu
from jax.experimental.pallas import tpu_sc as plsc

NUM_TOKENS = 16384
SEQ = 26
D = 64
NW = 32
TOK_PER_W = NUM_TOKENS // NW
IDX_PER_W = TOK_PER_W * SEQ
N_T1 = TOK_PER_W // 128
N_UNITS = SEQ * N_T1
NBUF = 2


def _make_kernel():
    mesh = plsc.VectorSubcoreMesh(core_axis_name="c", subcore_axis_name="s")

    @functools.partial(
        pl.kernel,
        mesh=mesh,
        out_type=jax.ShapeDtypeStruct((SEQ, 8, 128, 8, 128), jnp.float32),
        compiler_params=pltpu.CompilerParams(
            use_tc_tiling_on_sc=False, needs_layout_passes=False
        ),
        scratch_types=[
            pltpu.VMEM((IDX_PER_W,), jnp.int32),
            pltpu.VMEM((IDX_PER_W,), jnp.int32),
            pltpu.VMEM((NBUF, 128, D), jnp.float32),
            pltpu.VMEM((NBUF, D, 129), jnp.float32),
            pltpu.SemaphoreType.DMA((NBUF,)),
            pltpu.SemaphoreType.DMA((NBUF,)),
        ],
    )
    def k(idx_hbm, table_hbm, out_hbm, idx_tmp, idx_s, rows, tiles, gsem, osem):
        wid = lax.axis_index("s") * 2 + lax.axis_index("c")
        pltpu.sync_copy(idx_hbm.at[pl.ds(wid * IDX_PER_W, IDX_PER_W)], idx_tmp)

        iota = lax.broadcasted_iota(jnp.int32, (16,), 0)

        def rearr(g, carry):
            t0 = g * 16
            for s in range(SEQ):
                src = iota * SEQ + (t0 * SEQ + s)
                vals = plsc.load_gather(idx_tmp, [src])
                idx_s[pl.ds(s * TOK_PER_W + t0, 16)] = vals
            return carry

        lax.fori_loop(0, TOK_PER_W // 16, rearr, 0)

        def unit_off(u):
            s = u // N_T1
            t1l = lax.rem(u, N_T1)
            return s, t1l, s * TOK_PER_W + t1l * 128

        def gather(u, b):
            _, _, off = unit_off(u)
            return pltpu.make_async_copy(
                table_hbm.at[idx_s.at[pl.ds(off, 128)]],
                rows.at[b],
                gsem.at[b],
            )

        def out_copy(u, b, d1):
            s, t1l, _ = unit_off(u)
            t1g = wid * N_T1 + t1l
            return pltpu.make_async_copy(
                tiles.at[b, pl.ds(d1 * 8, 8), pl.ds(0, 128)],
                out_hbm.at[s, d1, t1g],
                osem.at[b],
            )

        def drain_outs(b):
            pltpu.make_async_copy(
                table_hbm.at[pl.ds(0, 128)], rows.at[b], osem.at[b]
            ).wait()

        d_base = [iota + 16 * kk for kk in range(4)]

        for b in range(NBUF):
            gather(b, b).start()

        def pair(p, carry):
            for b in range(NBUF):
                u = p * NBUF + b
                gather(u, b).wait()

                @pl.when(u >= NBUF)
                def _():
                    drain_outs(b)

                def trans(t2g, c2):
                    for tt in range(8):
                        t2 = t2g * 8 + tt
                        t2v = jnp.full((16,), 0, jnp.int32) + t2
                        for kk in range(4):
                            v = rows[b, t2, pl.ds(16 * kk, 16)]
                            plsc.store_scatter(
                                tiles.at[b], [d_base[kk], t2v], v
                            )
                    return c2

                lax.fori_loop(0, 16, trans, 0)

                @pl.when(u + NBUF < N_UNITS)
                def _():
                    gather(u + NBUF, b).start()

                for d1 in range(8):
                    out_copy(u, b, d1).start()
            return carry

        lax.fori_loop(0, N_UNITS // NBUF, pair, 0)

        for b in range(NBUF):
            drain_outs(b)

    return k


def kernel(token_ids, weights):
    idx = token_ids.astype(jnp.int32).reshape(-1)
    out5 = _make_kernel()(idx, weights)
    return out5.transpose(2, 4, 0, 1, 3).reshape(NUM_TOKENS, SEQ, D)
